# baseline (device time: 25288 ns/iter reference)
import jax
import jax.numpy as jnp
from jax import lax
from jax.experimental import pallas as pl
from jax.experimental.pallas import tpu as pltpu

E_LOC = 2


def kernel(x, router, W1, W2):
    t_loc, d = x.shape

    def body(x_ref, r_ref, w1_hbm, w2_hbm, out_ref,
             w1_vmem, w2_vmem, rr_buf, xsend, xrecv, wsend, wrecv,
             csend, crecv, w_sems, send_sems, recv_sems):
        my_x = lax.axis_index("x")
        my_y = lax.axis_index("y")
        nbr = (1 - my_x, my_y)

        w_copies = []
        for e in range(E_LOC):
            c1 = pltpu.make_async_copy(w1_hbm.at[e], w1_vmem.at[e],
                                       w_sems.at[e])
            c1.start()
            c2 = pltpu.make_async_copy(w2_hbm.at[e], w2_vmem.at[e],
                                       w_sems.at[E_LOC + e])
            c2.start()
            w_copies.append((c1, c2))

        barrier_sem = pltpu.get_barrier_semaphore()
        pl.semaphore_signal(barrier_sem, inc=1, device_id=nbr,
                            device_id_type=pl.DeviceIdType.MESH)
        pl.semaphore_wait(barrier_sem, 1)

        rdma_r = pltpu.make_async_remote_copy(
            src_ref=r_ref, dst_ref=rr_buf,
            send_sem=send_sems.at[1], recv_sem=recv_sems.at[1],
            device_id=nbr, device_id_type=pl.DeviceIdType.MESH)
        rdma_r.start()
        xsend[...] = x_ref[...].astype(jnp.bfloat16)
        rdma_x = pltpu.make_async_remote_copy(
            src_ref=xsend, dst_ref=xrecv,
            send_sem=send_sems.at[0], recv_sem=recv_sems.at[0],
            device_id=nbr, device_id_type=pl.DeviceIdType.MESH)
        rdma_x.start()

        x_my = x_ref[...]
        rdma_r.wait()
        g4 = jnp.concatenate(
            [jnp.dot(x_my, r_ref[...], preferred_element_type=jnp.float32),
             jnp.dot(x_my, rr_buf[...], preferred_element_type=jnp.float32)],
            axis=1)
        m1 = jnp.max(g4, axis=1, keepdims=True)
        m2 = jnp.max(jnp.where(g4 == m1, -1e30, g4), axis=1, keepdims=True)
        w4 = jnp.where(g4 >= m2, jnp.exp(g4 - m1) / (1.0 + jnp.exp(m2 - m1)),
                       0.0)
        wsend[...] = w4[:, E_LOC:]
        rdma_w = pltpu.make_async_remote_copy(
            src_ref=wsend, dst_ref=wrecv,
            send_sem=send_sems.at[3], recv_sem=recv_sems.at[3],
            device_id=nbr, device_id_type=pl.DeviceIdType.MESH)
        rdma_w.start()

        w_copies[0][0].wait()
        h_my0 = jnp.maximum(
            jnp.dot(x_my, w1_vmem[0], preferred_element_type=jnp.float32),
            0.0)
        w_copies[1][0].wait()
        h_my1 = jnp.maximum(
            jnp.dot(x_my, w1_vmem[1], preferred_element_type=jnp.float32),
            0.0)
        w_copies[0][1].wait()
        c_my0 = jnp.dot(h_my0, w2_vmem[0],
                        preferred_element_type=jnp.float32)

        rdma_x.wait()
        rdma_w.wait()

        x_nb = xrecv[...].astype(jnp.float32)
        w_nb = wrecv[...]
        h_nb0 = jnp.maximum(
            jnp.dot(x_nb, w1_vmem[0], preferred_element_type=jnp.float32),
            0.0)
        c_nb0 = jnp.dot(h_nb0, w2_vmem[0],
                        preferred_element_type=jnp.float32)
        h_nb1 = jnp.maximum(
            jnp.dot(x_nb, w1_vmem[1], preferred_element_type=jnp.float32),
            0.0)
        w_copies[1][1].wait()
        c_nb1 = jnp.dot(h_nb1, w2_vmem[1],
                        preferred_element_type=jnp.float32)
        csend[...] = (c_nb0 * w_nb[:, 0:1] + c_nb1 * w_nb[:, 1:2]).astype(
            jnp.bfloat16)
        rdma_c = pltpu.make_async_remote_copy(
            src_ref=csend, dst_ref=crecv,
            send_sem=send_sems.at[2], recv_sem=recv_sems.at[2],
            device_id=nbr, device_id_type=pl.DeviceIdType.MESH)
        rdma_c.start()

        c_my1 = jnp.dot(h_my1, w2_vmem[1],
                        preferred_element_type=jnp.float32)
        out_mine = c_my0 * w4[:, 0:1] + c_my1 * w4[:, 1:2]

        rdma_c.wait()
        out_ref[...] = out_mine + crecv[...].astype(jnp.float32)

    return pl.pallas_call(
        body,
        out_shape=jax.ShapeDtypeStruct((t_loc, d), jnp.float32),
        in_specs=[
            pl.BlockSpec(memory_space=pltpu.VMEM),
            pl.BlockSpec(memory_space=pltpu.VMEM),
            pl.BlockSpec(memory_space=pl.ANY),
            pl.BlockSpec(memory_space=pl.ANY),
        ],
        out_specs=pl.BlockSpec(memory_space=pltpu.VMEM),
        scratch_shapes=[
            pltpu.VMEM(W1.shape, jnp.float32),
            pltpu.VMEM(W2.shape, jnp.float32),
            pltpu.VMEM(router.shape, jnp.float32),
            pltpu.VMEM((t_loc, d), jnp.bfloat16),
            pltpu.VMEM((t_loc, d), jnp.bfloat16),
            pltpu.VMEM((t_loc, E_LOC), jnp.float32),
            pltpu.VMEM((t_loc, E_LOC), jnp.float32),
            pltpu.VMEM((t_loc, d), jnp.bfloat16),
            pltpu.VMEM((t_loc, d), jnp.bfloat16),
            pltpu.SemaphoreType.DMA((2 * E_LOC,)),
            pltpu.SemaphoreType.DMA((4,)),
            pltpu.SemaphoreType.DMA((4,)),
        ],
        compiler_params=pltpu.CompilerParams(collective_id=0),
    )(x, router, W1, W2)


# device time: 24085 ns/iter; 1.0499x vs baseline; 1.0499x over previous
import jax
import jax.numpy as jnp
from jax import lax
from jax.experimental import pallas as pl
from jax.experimental.pallas import tpu as pltpu

E_LOC = 2


def kernel(x, router, W1, W2):
    t_loc, d = x.shape
    f = W1.shape[2]

    def body(x_ref, r_ref, w1_hbm, w2_hbm, out_ref,
             w1_vmem, w2_vmem, rr_buf, xsend, xrecv, csend, crecv,
             w_sems, send_sems, recv_sems):
        my_x = lax.axis_index("x")
        my_y = lax.axis_index("y")
        nbr = (1 - my_x, my_y)

        w_copies = []
        for e in range(E_LOC):
            c1 = pltpu.make_async_copy(w1_hbm.at[e], w1_vmem.at[e],
                                       w_sems.at[e])
            c1.start()
            c2 = pltpu.make_async_copy(w2_hbm.at[e], w2_vmem.at[e],
                                       w_sems.at[E_LOC + e])
            c2.start()
            w_copies.append((c1, c2))

        barrier_sem = pltpu.get_barrier_semaphore()
        pl.semaphore_signal(barrier_sem, inc=1, device_id=nbr,
                            device_id_type=pl.DeviceIdType.MESH)
        pl.semaphore_wait(barrier_sem, 1)

        xsend[...] = x_ref[...].astype(jnp.bfloat16)
        rdma_x = pltpu.make_async_remote_copy(
            src_ref=xsend, dst_ref=xrecv,
            send_sem=send_sems.at[0], recv_sem=recv_sems.at[0],
            device_id=nbr, device_id_type=pl.DeviceIdType.MESH)
        rdma_x.start()
        rdma_r = pltpu.make_async_remote_copy(
            src_ref=r_ref, dst_ref=rr_buf,
            send_sem=send_sems.at[1], recv_sem=recv_sems.at[1],
            device_id=nbr, device_id_type=pl.DeviceIdType.MESH)
        rdma_r.start()

        def topk_weights(g_loc, g_rem):
            g4 = jnp.concatenate([g_loc, g_rem], axis=1)
            m1 = jnp.max(g4, axis=1, keepdims=True)
            m2 = jnp.max(jnp.where(g4 == m1, -1e30, g4), axis=1,
                         keepdims=True)
            denom = 1.0 + jnp.exp(m2 - m1)
            return [jnp.where(g_loc[:, e:e + 1] >= m2,
                              jnp.exp(g_loc[:, e:e + 1] - m1) / denom, 0.0)
                    for e in range(E_LOC)]

        x_my = x_ref[...]
        w_copies[0][0].wait()
        h_my0 = jnp.maximum(
            jnp.dot(x_my, w1_vmem[0], preferred_element_type=jnp.float32),
            0.0)
        w_copies[1][0].wait()
        h_my1 = jnp.maximum(
            jnp.dot(x_my, w1_vmem[1], preferred_element_type=jnp.float32),
            0.0)
        w_copies[0][1].wait()
        c_my0 = jnp.dot(h_my0, w2_vmem[0],
                        preferred_element_type=jnp.float32)

        rdma_r.wait()
        rdma_x.wait()

        x_nb = xrecv[...].astype(jnp.float32)
        g_nb_loc = jnp.dot(x_nb, r_ref[...],
                           preferred_element_type=jnp.float32)
        g_nb_rem = jnp.dot(x_nb, rr_buf[...],
                           preferred_element_type=jnp.float32)
        w_nb = topk_weights(g_nb_loc, g_nb_rem)
        h_nb0 = jnp.maximum(
            jnp.dot(x_nb, w1_vmem[0], preferred_element_type=jnp.float32),
            0.0)
        c_nb0 = jnp.dot(h_nb0, w2_vmem[0],
                        preferred_element_type=jnp.float32)
        h_nb1 = jnp.maximum(
            jnp.dot(x_nb, w1_vmem[1], preferred_element_type=jnp.float32),
            0.0)
        w_copies[1][1].wait()
        c_nb1 = jnp.dot(h_nb1, w2_vmem[1],
                        preferred_element_type=jnp.float32)
        csend[...] = (c_nb0 * w_nb[0] + c_nb1 * w_nb[1]).astype(
            jnp.bfloat16)
        rdma_c = pltpu.make_async_remote_copy(
            src_ref=csend, dst_ref=crecv,
            send_sem=send_sems.at[2], recv_sem=recv_sems.at[2],
            device_id=nbr, device_id_type=pl.DeviceIdType.MESH)
        rdma_c.start()

        c_my1 = jnp.dot(h_my1, w2_vmem[1],
                        preferred_element_type=jnp.float32)
        g_my_loc = jnp.dot(x_my, r_ref[...],
                           preferred_element_type=jnp.float32)
        g_my_rem = jnp.dot(x_my, rr_buf[...],
                           preferred_element_type=jnp.float32)
        w_my = topk_weights(g_my_loc, g_my_rem)
        out_mine = c_my0 * w_my[0] + c_my1 * w_my[1]

        rdma_c.wait()
        out_ref[...] = out_mine + crecv[...].astype(jnp.float32)

    return pl.pallas_call(
        body,
        out_shape=jax.ShapeDtypeStruct((t_loc, d), jnp.float32),
        in_specs=[
            pl.BlockSpec(memory_space=pltpu.VMEM),
            pl.BlockSpec(memory_space=pltpu.VMEM),
            pl.BlockSpec(memory_space=pl.ANY),
            pl.BlockSpec(memory_space=pl.ANY),
        ],
        out_specs=pl.BlockSpec(memory_space=pltpu.VMEM),
        scratch_shapes=[
            pltpu.VMEM(W1.shape, jnp.float32),
            pltpu.VMEM(W2.shape, jnp.float32),
            pltpu.VMEM(router.shape, jnp.float32),
            pltpu.VMEM((t_loc, d), jnp.bfloat16),
            pltpu.VMEM((t_loc, d), jnp.bfloat16),
            pltpu.VMEM((t_loc, d), jnp.bfloat16),
            pltpu.VMEM((t_loc, d), jnp.bfloat16),
            pltpu.SemaphoreType.DMA((2 * E_LOC,)),
            pltpu.SemaphoreType.DMA((3,)),
            pltpu.SemaphoreType.DMA((3,)),
        ],
        compiler_params=pltpu.CompilerParams(collective_id=0),
    )(x, router, W1, W2)


# device time: 19249 ns/iter; 1.3137x vs baseline; 1.2512x over previous
import jax
import jax.numpy as jnp
from jax import lax
from jax.experimental import pallas as pl
from jax.experimental.pallas import tpu as pltpu

E_LOC = 2


def kernel(x, router, W1, W2):
    t_loc, d = x.shape

    x = pltpu.with_memory_space_constraint(x, pltpu.MemorySpace.HBM)
    W1 = pltpu.with_memory_space_constraint(W1, pltpu.MemorySpace.HBM)
    W2 = pltpu.with_memory_space_constraint(W2, pltpu.MemorySpace.HBM)

    def body(x_hbm, r_ref, w1_hbm, w2_hbm, out_ref,
             x_vmem, w1_vmem, w2_vmem, rr_buf, xsend, xrecv, csend, crecv,
             x_sem, w_sems, send_sems, recv_sems):
        my_x = lax.axis_index("x")
        my_y = lax.axis_index("y")
        nbr = (1 - my_x, my_y)

        x_copy = pltpu.make_async_copy(x_hbm, x_vmem, x_sem)
        x_copy.start()
        w_copies = []
        for e in range(E_LOC):
            c1 = pltpu.make_async_copy(w1_hbm.at[e], w1_vmem.at[e],
                                       w_sems.at[e])
            c1.start()
            c2 = pltpu.make_async_copy(w2_hbm.at[e], w2_vmem.at[e],
                                       w_sems.at[E_LOC + e])
            c2.start()
            w_copies.append((c1, c2))

        barrier_sem = pltpu.get_barrier_semaphore()
        pl.semaphore_signal(barrier_sem, inc=1, device_id=nbr,
                            device_id_type=pl.DeviceIdType.MESH)
        pl.semaphore_wait(barrier_sem, 1)

        x_copy.wait()
        xsend[...] = x_vmem[...].astype(jnp.bfloat16)
        rdma_x = pltpu.make_async_remote_copy(
            src_ref=xsend, dst_ref=xrecv,
            send_sem=send_sems.at[0], recv_sem=recv_sems.at[0],
            device_id=nbr, device_id_type=pl.DeviceIdType.MESH)
        rdma_x.start()
        rdma_r = pltpu.make_async_remote_copy(
            src_ref=r_ref, dst_ref=rr_buf,
            send_sem=send_sems.at[1], recv_sem=recv_sems.at[1],
            device_id=nbr, device_id_type=pl.DeviceIdType.MESH)
        rdma_r.start()

        def topk_weights(g_loc, g_rem):
            g4 = jnp.concatenate([g_loc, g_rem], axis=1)
            m1 = jnp.max(g4, axis=1, keepdims=True)
            m2 = jnp.max(jnp.where(g4 == m1, -1e30, g4), axis=1,
                         keepdims=True)
            denom = 1.0 + jnp.exp(m2 - m1)
            return [jnp.where(g_loc[:, e:e + 1] >= m2,
                              jnp.exp(g_loc[:, e:e + 1] - m1) / denom, 0.0)
                    for e in range(E_LOC)]

        x_my = x_vmem[...]
        w_copies[0][0].wait()
        h_my0 = jnp.maximum(
            jnp.dot(x_my, w1_vmem[0], preferred_element_type=jnp.float32),
            0.0)
        w_copies[1][0].wait()
        h_my1 = jnp.maximum(
            jnp.dot(x_my, w1_vmem[1], preferred_element_type=jnp.float32),
            0.0)
        w_copies[0][1].wait()
        c_my0 = jnp.dot(h_my0, w2_vmem[0],
                        preferred_element_type=jnp.float32)

        rdma_r.wait()
        rdma_x.wait()

        x_nb = xrecv[...].astype(jnp.float32)
        g_nb_loc = jnp.dot(x_nb, r_ref[...],
                           preferred_element_type=jnp.float32)
        g_nb_rem = jnp.dot(x_nb, rr_buf[...],
                           preferred_element_type=jnp.float32)
        w_nb = topk_weights(g_nb_loc, g_nb_rem)
        h_nb0 = jnp.maximum(
            jnp.dot(x_nb, w1_vmem[0], preferred_element_type=jnp.float32),
            0.0)
        c_nb0 = jnp.dot(h_nb0, w2_vmem[0],
                        preferred_element_type=jnp.float32)
        h_nb1 = jnp.maximum(
            jnp.dot(x_nb, w1_vmem[1], preferred_element_type=jnp.float32),
            0.0)
        w_copies[1][1].wait()
        c_nb1 = jnp.dot(h_nb1, w2_vmem[1],
                        preferred_element_type=jnp.float32)
        csend[...] = (c_nb0 * w_nb[0] + c_nb1 * w_nb[1]).astype(
            jnp.bfloat16)
        rdma_c = pltpu.make_async_remote_copy(
            src_ref=csend, dst_ref=crecv,
            send_sem=send_sems.at[2], recv_sem=recv_sems.at[2],
            device_id=nbr, device_id_type=pl.DeviceIdType.MESH)
        rdma_c.start()

        c_my1 = jnp.dot(h_my1, w2_vmem[1],
                        preferred_element_type=jnp.float32)
        g_my_loc = jnp.dot(x_my, r_ref[...],
                           preferred_element_type=jnp.float32)
        g_my_rem = jnp.dot(x_my, rr_buf[...],
                           preferred_element_type=jnp.float32)
        w_my = topk_weights(g_my_loc, g_my_rem)
        out_mine = c_my0 * w_my[0] + c_my1 * w_my[1]

        rdma_c.wait()
        out_ref[...] = out_mine + crecv[...].astype(jnp.float32)

    return pl.pallas_call(
        body,
        out_shape=jax.ShapeDtypeStruct((t_loc, d), jnp.float32),
        in_specs=[
            pl.BlockSpec(memory_space=pltpu.MemorySpace.HBM),
            pl.BlockSpec(memory_space=pltpu.VMEM),
            pl.BlockSpec(memory_space=pltpu.MemorySpace.HBM),
            pl.BlockSpec(memory_space=pltpu.MemorySpace.HBM),
        ],
        out_specs=pl.BlockSpec(memory_space=pltpu.VMEM),
        scratch_shapes=[
            pltpu.VMEM((t_loc, d), jnp.float32),
            pltpu.VMEM(W1.shape, jnp.float32),
            pltpu.VMEM(W2.shape, jnp.float32),
            pltpu.VMEM(router.shape, jnp.float32),
            pltpu.VMEM((t_loc, d), jnp.bfloat16),
            pltpu.VMEM((t_loc, d), jnp.bfloat16),
            pltpu.VMEM((t_loc, d), jnp.bfloat16),
            pltpu.VMEM((t_loc, d), jnp.bfloat16),
            pltpu.SemaphoreType.DMA,
            pltpu.SemaphoreType.DMA((2 * E_LOC,)),
            pltpu.SemaphoreType.DMA((3,)),
            pltpu.SemaphoreType.DMA((3,)),
        ],
        compiler_params=pltpu.CompilerParams(collective_id=0),
    )(x, router, W1, W2)


# device time: 18530 ns/iter; 1.3647x vs baseline; 1.0388x over previous
import jax
import jax.numpy as jnp
from jax import lax
from jax.experimental import pallas as pl
from jax.experimental.pallas import tpu as pltpu

E_LOC = 2
N_CHUNK = 2


def kernel(x, router, W1, W2):
    t_loc, d = x.shape
    t_ch = t_loc // N_CHUNK

    x = pltpu.with_memory_space_constraint(x, pltpu.MemorySpace.HBM)
    W1 = pltpu.with_memory_space_constraint(W1, pltpu.MemorySpace.HBM)
    W2 = pltpu.with_memory_space_constraint(W2, pltpu.MemorySpace.HBM)

    def body(x_hbm, r_ref, w1_hbm, w2_hbm, out_ref,
             x_vmem, w1_vmem, w2_vmem, rr_buf, xsend, xrecv, csend, crecv,
             x_sem, w_sems, xr_send_sems, xr_recv_sems,
             c_send_sems, c_recv_sems):
        my_x = lax.axis_index("x")
        my_y = lax.axis_index("y")
        nbr = (1 - my_x, my_y)

        x_copy = pltpu.make_async_copy(x_hbm, x_vmem, x_sem)
        x_copy.start()
        w_copies = []
        for e in range(E_LOC):
            c1 = pltpu.make_async_copy(w1_hbm.at[e], w1_vmem.at[e],
                                       w_sems.at[e])
            c1.start()
            c2 = pltpu.make_async_copy(w2_hbm.at[e], w2_vmem.at[e],
                                       w_sems.at[E_LOC + e])
            c2.start()
            w_copies.append((c1, c2))

        barrier_sem = pltpu.get_barrier_semaphore()
        pl.semaphore_signal(barrier_sem, inc=1, device_id=nbr,
                            device_id_type=pl.DeviceIdType.MESH)
        pl.semaphore_wait(barrier_sem, 1)

        x_copy.wait()
        xsend[...] = x_vmem[...].astype(jnp.bfloat16)
        rdma_x = []
        for ch in range(N_CHUNK):
            sl = pl.ds(ch * t_ch, t_ch)
            r = pltpu.make_async_remote_copy(
                src_ref=xsend.at[sl], dst_ref=xrecv.at[sl],
                send_sem=xr_send_sems.at[ch], recv_sem=xr_recv_sems.at[ch],
                device_id=nbr, device_id_type=pl.DeviceIdType.MESH)
            r.start()
            rdma_x.append(r)
        rdma_r = pltpu.make_async_remote_copy(
            src_ref=r_ref, dst_ref=rr_buf,
            send_sem=xr_send_sems.at[N_CHUNK], recv_sem=xr_recv_sems.at[N_CHUNK],
            device_id=nbr, device_id_type=pl.DeviceIdType.MESH)
        rdma_r.start()

        def topk_weights(g_loc, g_rem):
            g4 = jnp.concatenate([g_loc, g_rem], axis=1)
            m1 = jnp.max(g4, axis=1, keepdims=True)
            m2 = jnp.max(jnp.where(g4 == m1, -1e30, g4), axis=1,
                         keepdims=True)
            denom = 1.0 + jnp.exp(m2 - m1)
            return [jnp.where(g_loc[:, e:e + 1] >= m2,
                              jnp.exp(g_loc[:, e:e + 1] - m1) / denom, 0.0)
                    for e in range(E_LOC)]

        x_my = x_vmem[...]
        w_copies[0][0].wait()
        h_my0 = jnp.maximum(
            jnp.dot(x_my, w1_vmem[0], preferred_element_type=jnp.float32),
            0.0)
        w_copies[1][0].wait()
        h_my1 = jnp.maximum(
            jnp.dot(x_my, w1_vmem[1], preferred_element_type=jnp.float32),
            0.0)
        w_copies[0][1].wait()
        w_copies[1][1].wait()

        rdma_r.wait()

        rdma_c = []
        for ch in range(N_CHUNK):
            sl = pl.ds(ch * t_ch, t_ch)
            rdma_x[ch].wait()
            x_nb = xrecv[sl, :].astype(jnp.float32)
            g_nb_loc = jnp.dot(x_nb, r_ref[...],
                               preferred_element_type=jnp.float32)
            g_nb_rem = jnp.dot(x_nb, rr_buf[...],
                               preferred_element_type=jnp.float32)
            w_nb = topk_weights(g_nb_loc, g_nb_rem)
            h_nb0 = jnp.maximum(
                jnp.dot(x_nb, w1_vmem[0],
                        preferred_element_type=jnp.float32), 0.0)
            c_nb0 = jnp.dot(h_nb0, w2_vmem[0],
                            preferred_element_type=jnp.float32)
            h_nb1 = jnp.maximum(
                jnp.dot(x_nb, w1_vmem[1],
                        preferred_element_type=jnp.float32), 0.0)
            c_nb1 = jnp.dot(h_nb1, w2_vmem[1],
                            preferred_element_type=jnp.float32)
            csend[sl, :] = (c_nb0 * w_nb[0] + c_nb1 * w_nb[1]).astype(
                jnp.bfloat16)
            r = pltpu.make_async_remote_copy(
                src_ref=csend.at[sl], dst_ref=crecv.at[sl],
                send_sem=c_send_sems.at[ch], recv_sem=c_recv_sems.at[ch],
                device_id=nbr, device_id_type=pl.DeviceIdType.MESH)
            r.start()
            rdma_c.append(r)

        c_my0 = jnp.dot(h_my0, w2_vmem[0],
                        preferred_element_type=jnp.float32)
        c_my1 = jnp.dot(h_my1, w2_vmem[1],
                        preferred_element_type=jnp.float32)
        g_my_loc = jnp.dot(x_my, r_ref[...],
                           preferred_element_type=jnp.float32)
        g_my_rem = jnp.dot(x_my, rr_buf[...],
                           preferred_element_type=jnp.float32)
        w_my = topk_weights(g_my_loc, g_my_rem)
        out_mine = c_my0 * w_my[0] + c_my1 * w_my[1]

        for ch in range(N_CHUNK):
            rdma_c[ch].wait()
        out_ref[...] = out_mine + crecv[...].astype(jnp.float32)

    return pl.pallas_call(
        body,
        out_shape=jax.ShapeDtypeStruct((t_loc, d), jnp.float32),
        in_specs=[
            pl.BlockSpec(memory_space=pltpu.MemorySpace.HBM),
            pl.BlockSpec(memory_space=pltpu.VMEM),
            pl.BlockSpec(memory_space=pltpu.MemorySpace.HBM),
            pl.BlockSpec(memory_space=pltpu.MemorySpace.HBM),
        ],
        out_specs=pl.BlockSpec(memory_space=pltpu.VMEM),
        scratch_shapes=[
            pltpu.VMEM((t_loc, d), jnp.float32),
            pltpu.VMEM(W1.shape, jnp.float32),
            pltpu.VMEM(W2.shape, jnp.float32),
            pltpu.VMEM(router.shape, jnp.float32),
            pltpu.VMEM((t_loc, d), jnp.bfloat16),
            pltpu.VMEM((t_loc, d), jnp.bfloat16),
            pltpu.VMEM((t_loc, d), jnp.bfloat16),
            pltpu.VMEM((t_loc, d), jnp.bfloat16),
            pltpu.SemaphoreType.DMA,
            pltpu.SemaphoreType.DMA((2 * E_LOC,)),
            pltpu.SemaphoreType.DMA((N_CHUNK + 1,)),
            pltpu.SemaphoreType.DMA((N_CHUNK + 1,)),
            pltpu.SemaphoreType.DMA((N_CHUNK,)),
            pltpu.SemaphoreType.DMA((N_CHUNK,)),
        ],
        compiler_params=pltpu.CompilerParams(collective_id=0),
    )(x, router, W1, W2)
